# Initial kernel scaffold; baseline (speedup 1.0000x reference)
import jax
import jax.numpy as jnp
from jax.experimental import pallas as pl


def kernel(qk, v):
    raise NotImplementedError("write your pallas kernel here")


# speedup vs baseline: 5.6205x; 5.6205x over previous
import functools

import jax
import jax.numpy as jnp
from jax import lax
from jax.experimental import pallas as pl
from jax.experimental.pallas import tpu as pltpu
from jax.experimental.pallas import tpu_sc as plsc

BATCH = 8
SEQ = 4096
DIM = 64
N_HASHES = 8
BUCKET_SIZE = 64
N_BUCKETS = SEQ // BUCKET_SIZE
N_CHUNKS = SEQ // BUCKET_SIZE
TASKS = BATCH * N_HASHES
NEG_SELF = -10000.0
GCHUNK = 128


def _hash_sort_body(qk_ref, rot_ref, bucket_ref, pos_ref):
    qk = qk_ref[0]
    rot = rot_ref[0]
    r = lax.dot_general(qk, rot, (((1,), (0,)), ((), ())),
                        preferred_element_type=jnp.float32)
    r2 = jnp.concatenate([r, -r], axis=-1)
    m = jnp.max(r2, axis=-1, keepdims=True)
    col = lax.broadcasted_iota(jnp.int32, r2.shape, 1)
    bucket = jnp.min(jnp.where(r2 == m, col, N_BUCKETS), axis=-1,
                     keepdims=True)

    onehot = (bucket == lax.broadcasted_iota(
        jnp.int32, (SEQ, N_BUCKETS), 1)).astype(jnp.float32)

    cum = onehot
    k = 1
    while k < SEQ:
        cum = cum + jnp.concatenate(
            [jnp.zeros((k, N_BUCKETS), jnp.float32), cum[:-k]], axis=0)
        k *= 2

    counts = jnp.sum(onehot, axis=0, keepdims=True)
    ci = counts.astype(jnp.int32)
    hi = (ci >> 8).astype(jnp.float32)
    lo = (ci & 255).astype(jnp.float32)
    tri = (lax.broadcasted_iota(jnp.int32, (N_BUCKETS, N_BUCKETS), 0)
           < lax.broadcasted_iota(
               jnp.int32, (N_BUCKETS, N_BUCKETS), 1)).astype(jnp.float32)
    off = (lax.dot_general(hi, tri, (((1,), (0,)), ((), ()))) * 256.0
           + lax.dot_general(lo, tri, (((1,), (0,)), ((), ()))))

    pos = jnp.sum(onehot * (cum - 1.0 + off), axis=-1, keepdims=True)
    bucket_ref[0] = bucket
    pos_ref[0] = pos.astype(jnp.int32)


def _hash_sort(qk, rot_t):
    return pl.pallas_call(
        _hash_sort_body,
        grid=(BATCH, N_HASHES),
        in_specs=[
            pl.BlockSpec((1, SEQ, DIM), lambda b, h: (b, 0, 0)),
            pl.BlockSpec((1, DIM, N_BUCKETS // 2), lambda b, h: (h, 0, 0)),
        ],
        out_specs=[
            pl.BlockSpec((1, SEQ, 1), lambda b, h: (b * N_HASHES + h, 0, 0)),
            pl.BlockSpec((1, SEQ, 1), lambda b, h: (b * N_HASHES + h, 0, 0)),
        ],
        out_shape=[
            jax.ShapeDtypeStruct((TASKS, SEQ, 1), jnp.int32),
            jax.ShapeDtypeStruct((TASKS, SEQ, 1), jnp.int32),
        ],
    )(qk, rot_t)


def _permute_body(qk_hbm, v_hbm, pos_hbm, bkt_hbm,
                  sqk_hbm, sv_hbm, st_hbm, sbk_hbm,
                  pos_v, bkt_v, st_v, sbk_v, idxg_v, bufq, bufv, sem):
    nc = 2
    wid = lax.axis_index("s") * nc + lax.axis_index("c")

    def one_task(r, _):
        t = wid * 2 + r
        b = t // N_HASHES
        toff = pl.multiple_of(t * SEQ, SEQ)
        pltpu.sync_copy(pos_hbm.at[pl.ds(toff, SEQ)], pos_v)
        pltpu.sync_copy(bkt_hbm.at[pl.ds(toff, SEQ)], bkt_v)

        boff = b * SEQ

        def scat(j, _):
            base = pl.multiple_of(j * 16, 16)
            idx = pos_v[pl.ds(base, 16)]
            tok = lax.iota(jnp.int32, 16) + base
            plsc.store_scatter(st_v, [idx], tok)
            plsc.store_scatter(idxg_v, [idx], tok + boff)
            plsc.store_scatter(sbk_v, [idx], bkt_v[pl.ds(base, 16)])
            return 0

        lax.fori_loop(0, SEQ // 16, scat, 0)

        pltpu.sync_copy(st_v, st_hbm.at[pl.ds(toff, SEQ)])
        pltpu.sync_copy(sbk_v, sbk_hbm.at[pl.ds(toff, SEQ)])

        def gat(c, _):
            goff = pl.multiple_of(c * GCHUNK, GCHUNK)
            idxs = idxg_v.at[pl.ds(goff, GCHUNK)]
            pltpu.async_copy(qk_hbm.at[idxs], bufq, sem).wait()
            pltpu.sync_copy(bufq, sqk_hbm.at[pl.ds(toff + goff, GCHUNK)])
            pltpu.async_copy(v_hbm.at[idxs], bufv, sem).wait()
            pltpu.sync_copy(bufv, sv_hbm.at[pl.ds(toff + goff, GCHUNK)])
            return 0

        lax.fori_loop(0, SEQ // GCHUNK, gat, 0)
        return 0

    lax.fori_loop(0, TASKS // 32, one_task, 0)


def _permute(qk_flat, v_flat, pos_flat, bkt_flat):
    tot = TASKS * SEQ
    mesh = plsc.VectorSubcoreMesh(core_axis_name="c", subcore_axis_name="s")
    fn = functools.partial(
        pl.kernel,
        out_type=[
            jax.ShapeDtypeStruct((tot, DIM), jnp.float32),
            jax.ShapeDtypeStruct((tot, DIM), jnp.float32),
            jax.ShapeDtypeStruct((tot,), jnp.int32),
            jax.ShapeDtypeStruct((tot,), jnp.int32),
        ],
        mesh=mesh,
        compiler_params=pltpu.CompilerParams(
            needs_layout_passes=False, use_tc_tiling_on_sc=False),
        scratch_types=[
            pltpu.VMEM((SEQ,), jnp.int32),
            pltpu.VMEM((SEQ,), jnp.int32),
            pltpu.VMEM((SEQ,), jnp.int32),
            pltpu.VMEM((SEQ,), jnp.int32),
            pltpu.VMEM((SEQ,), jnp.int32),
            pltpu.VMEM((GCHUNK, DIM), jnp.float32),
            pltpu.VMEM((GCHUNK, DIM), jnp.float32),
            pltpu.SemaphoreType.DMA,
        ],
    )(_permute_body)
    return fn(qk_flat, v_flat, pos_flat, bkt_flat)


def _attend_body(sqk_ref, sv_ref, stq_ref, stk_ref, sbq_ref, sbkk_ref,
                 so_ref, slse_ref):
    sqk = sqk_ref[0]
    sv = sv_ref[0]
    stq = stq_ref[0].reshape(N_CHUNKS, BUCKET_SIZE, 1)
    sbq = sbq_ref[0].reshape(N_CHUNKS, BUCKET_SIZE, 1)
    tk = stk_ref[0]
    bkk = sbkk_ref[0]

    norms = jnp.sqrt(jnp.sum(sqk * sqk, axis=-1, keepdims=True))
    kn = sqk / jnp.maximum(norms, 1e-12)

    q = sqk.reshape(N_CHUNKS, BUCKET_SIZE, DIM)
    kc = kn.reshape(N_CHUNKS, BUCKET_SIZE, DIM)
    kwin = jnp.concatenate(
        [kc, jnp.concatenate([kc[-1:], kc[:-1]], axis=0)], axis=1)
    vc = sv.reshape(N_CHUNKS, BUCKET_SIZE, DIM)
    vwin = jnp.concatenate(
        [vc, jnp.concatenate([vc[-1:], vc[:-1]], axis=0)], axis=1)
    twin = jnp.concatenate(
        [tk, jnp.concatenate([tk[-1:], tk[:-1]], axis=0)],
        axis=1).reshape(N_CHUNKS, 1, 2 * BUCKET_SIZE)
    bwin = jnp.concatenate(
        [bkk, jnp.concatenate([bkk[-1:], bkk[:-1]], axis=0)],
        axis=1).reshape(N_CHUNKS, 1, 2 * BUCKET_SIZE)

    dots = lax.dot_general(q, kwin, (((2,), (2,)), ((0,), (0,))))
    dots = dots * (DIM ** -0.5)
    dots = jnp.where(stq == twin, NEG_SELF, dots)
    ch = lax.broadcasted_iota(jnp.int32, dots.shape, 0)
    zi = lax.broadcasted_iota(jnp.int32, dots.shape, 2)
    maskm = (sbq != bwin) | ((ch == 0) & (zi >= BUCKET_SIZE))
    dots = jnp.where(maskm, -jnp.finfo(jnp.float32).max, dots)

    m = jnp.max(dots, axis=-1, keepdims=True)
    s = jnp.sum(jnp.exp(dots - m), axis=-1, keepdims=True)
    lse = m + jnp.log(s)
    p = jnp.exp(dots - lse)
    bo = lax.dot_general(p, vwin, (((2,), (1,)), ((0,), (0,))))
    so_ref[0] = bo.reshape(SEQ, DIM)
    slse_ref[0] = lse.reshape(SEQ, 1)


def _attend(sqk, sv, stq, stk, sbq, sbkk):
    return pl.pallas_call(
        _attend_body,
        grid=(TASKS,),
        in_specs=[
            pl.BlockSpec((1, SEQ, DIM), lambda t: (t, 0, 0)),
            pl.BlockSpec((1, SEQ, DIM), lambda t: (t, 0, 0)),
            pl.BlockSpec((1, SEQ, 1), lambda t: (t, 0, 0)),
            pl.BlockSpec((1, N_CHUNKS, BUCKET_SIZE), lambda t: (t, 0, 0)),
            pl.BlockSpec((1, SEQ, 1), lambda t: (t, 0, 0)),
            pl.BlockSpec((1, N_CHUNKS, BUCKET_SIZE), lambda t: (t, 0, 0)),
        ],
        out_specs=[
            pl.BlockSpec((1, SEQ, DIM), lambda t: (t, 0, 0)),
            pl.BlockSpec((1, SEQ, 1), lambda t: (t, 0, 0)),
        ],
        out_shape=[
            jax.ShapeDtypeStruct((TASKS, SEQ, DIM), jnp.float32),
            jax.ShapeDtypeStruct((TASKS, SEQ, 1), jnp.float32),
        ],
    )(sqk, sv, stq, stk, sbq, sbkk)


def _unpermute_body(so_hbm, slse_hbm, pos_hbm,
                    o_hbm, lg_hbm,
                    pos_v, gidx_v, lse_v, lgo_v, bufo, sem):
    nc = 2
    wid = lax.axis_index("s") * nc + lax.axis_index("c")

    def one_task(r, _):
        t = wid * 2 + r
        toff = pl.multiple_of(t * SEQ, SEQ)
        pltpu.sync_copy(pos_hbm.at[pl.ds(toff, SEQ)], pos_v)
        pltpu.sync_copy(slse_hbm.at[pl.ds(toff, SEQ)], lse_v)

        def addoff(j, _):
            base = pl.multiple_of(j * 16, 16)
            idx = pos_v[pl.ds(base, 16)]
            gidx_v[pl.ds(base, 16)] = idx + toff
            lgo_v[pl.ds(base, 16)] = plsc.load_gather(lse_v, [idx])
            return 0

        lax.fori_loop(0, SEQ // 16, addoff, 0)
        pltpu.sync_copy(lgo_v, lg_hbm.at[pl.ds(toff, SEQ)])

        def gat(c, _):
            goff = pl.multiple_of(c * GCHUNK, GCHUNK)
            idxs = gidx_v.at[pl.ds(goff, GCHUNK)]
            pltpu.async_copy(so_hbm.at[idxs], bufo, sem).wait()
            pltpu.sync_copy(bufo, o_hbm.at[pl.ds(toff + goff, GCHUNK)])
            return 0

        lax.fori_loop(0, SEQ // GCHUNK, gat, 0)
        return 0

    lax.fori_loop(0, TASKS // 32, one_task, 0)


def _unpermute(so_flat, slse_flat, pos_flat):
    tot = TASKS * SEQ
    mesh = plsc.VectorSubcoreMesh(core_axis_name="c", subcore_axis_name="s")
    fn = functools.partial(
        pl.kernel,
        out_type=[
            jax.ShapeDtypeStruct((tot, DIM), jnp.float32),
            jax.ShapeDtypeStruct((tot,), jnp.float32),
        ],
        mesh=mesh,
        compiler_params=pltpu.CompilerParams(
            needs_layout_passes=False, use_tc_tiling_on_sc=False),
        scratch_types=[
            pltpu.VMEM((SEQ,), jnp.int32),
            pltpu.VMEM((SEQ,), jnp.int32),
            pltpu.VMEM((SEQ,), jnp.float32),
            pltpu.VMEM((SEQ,), jnp.float32),
            pltpu.VMEM((GCHUNK, DIM), jnp.float32),
            pltpu.SemaphoreType.DMA,
        ],
    )(_unpermute_body)
    return fn(so_flat, slse_flat, pos_flat)


def _combine_body(o_ref, lg_ref, out_ref):
    o = o_ref[0]
    lg = lg_ref[0]
    m = jnp.max(lg, axis=0, keepdims=True)
    s = jnp.sum(jnp.exp(lg - m), axis=0, keepdims=True)
    lse = m + jnp.log(s)
    p = jnp.exp(lg - lse)
    out_ref[0] = jnp.sum(o * p, axis=0)


_CSEQ = 512


def _combine(o4, lg4):
    return pl.pallas_call(
        _combine_body,
        grid=(BATCH, SEQ // _CSEQ),
        in_specs=[
            pl.BlockSpec((1, N_HASHES, _CSEQ, DIM), lambda b, s: (b, 0, s, 0)),
            pl.BlockSpec((1, N_HASHES, _CSEQ, 1), lambda b, s: (b, 0, s, 0)),
        ],
        out_specs=pl.BlockSpec((1, _CSEQ, DIM), lambda b, s: (b, s, 0)),
        out_shape=jax.ShapeDtypeStruct((BATCH, SEQ, DIM), jnp.float32),
    )(o4, lg4)


def kernel(qk, v):
    rot = jax.random.normal(jax.random.key(42),
                            (DIM, N_HASHES, N_BUCKETS // 2), dtype=qk.dtype)
    rot_t = jnp.transpose(rot, (1, 0, 2))

    bucket, pos = _hash_sort(qk, rot_t)
    pos_flat = pos.reshape(TASKS * SEQ)
    bkt_flat = bucket.reshape(TASKS * SEQ)

    qk_flat = qk.reshape(BATCH * SEQ, DIM)
    v_flat = v.reshape(BATCH * SEQ, DIM)
    sqk_flat, sv_flat, st_flat, sbk_flat = _permute(
        qk_flat, v_flat, pos_flat, bkt_flat)

    so, slse = _attend(sqk_flat.reshape(TASKS, SEQ, DIM),
                       sv_flat.reshape(TASKS, SEQ, DIM),
                       st_flat.reshape(TASKS, SEQ, 1),
                       st_flat.reshape(TASKS, N_CHUNKS, BUCKET_SIZE),
                       sbk_flat.reshape(TASKS, SEQ, 1),
                       sbk_flat.reshape(TASKS, N_CHUNKS, BUCKET_SIZE))

    o_flat, lg_flat = _unpermute(so.reshape(TASKS * SEQ, DIM),
                                 slse.reshape(TASKS * SEQ),
                                 pos_flat)

    out = _combine(o_flat.reshape(BATCH, N_HASHES, SEQ, DIM),
                   lg_flat.reshape(BATCH, N_HASHES, SEQ, 1))
    return out


# speedup vs baseline: 5.6246x; 1.0007x over previous
import functools

import jax
import jax.numpy as jnp
from jax import lax
from jax.experimental import pallas as pl
from jax.experimental.pallas import tpu as pltpu
from jax.experimental.pallas import tpu_sc as plsc

BATCH = 8
SEQ = 4096
DIM = 64
N_HASHES = 8
BUCKET_SIZE = 64
N_BUCKETS = SEQ // BUCKET_SIZE
N_CHUNKS = SEQ // BUCKET_SIZE
TASKS = BATCH * N_HASHES
NEG_SELF = -10000.0
GCHUNK = 128


def _hash_sort_body(qk_ref, rot_ref, bucket_ref, pos_ref):
    qk = qk_ref[0]
    rot = rot_ref[0]
    r = lax.dot_general(qk, rot, (((1,), (0,)), ((), ())),
                        preferred_element_type=jnp.float32)
    r2 = jnp.concatenate([r, -r], axis=-1)
    m = jnp.max(r2, axis=-1, keepdims=True)
    col = lax.broadcasted_iota(jnp.int32, r2.shape, 1)
    bucket = jnp.min(jnp.where(r2 == m, col, N_BUCKETS), axis=-1,
                     keepdims=True)

    onehot = (bucket == lax.broadcasted_iota(
        jnp.int32, (SEQ, N_BUCKETS), 1)).astype(jnp.float32)

    cum = onehot
    k = 1
    while k < SEQ:
        cum = cum + jnp.concatenate(
            [jnp.zeros((k, N_BUCKETS), jnp.float32), cum[:-k]], axis=0)
        k *= 2

    counts = jnp.sum(onehot, axis=0, keepdims=True)
    ci = counts.astype(jnp.int32)
    hi = (ci >> 8).astype(jnp.float32)
    lo = (ci & 255).astype(jnp.float32)
    tri = (lax.broadcasted_iota(jnp.int32, (N_BUCKETS, N_BUCKETS), 0)
           < lax.broadcasted_iota(
               jnp.int32, (N_BUCKETS, N_BUCKETS), 1)).astype(jnp.float32)
    off = (lax.dot_general(hi, tri, (((1,), (0,)), ((), ()))) * 256.0
           + lax.dot_general(lo, tri, (((1,), (0,)), ((), ()))))

    pos = jnp.sum(onehot * (cum - 1.0 + off), axis=-1, keepdims=True)
    bucket_ref[0] = bucket
    pos_ref[0] = pos.astype(jnp.int32)


def _hash_sort(qk, rot_t):
    return pl.pallas_call(
        _hash_sort_body,
        grid=(BATCH, N_HASHES),
        in_specs=[
            pl.BlockSpec((1, SEQ, DIM), lambda b, h: (b, 0, 0)),
            pl.BlockSpec((1, DIM, N_BUCKETS // 2), lambda b, h: (h, 0, 0)),
        ],
        out_specs=[
            pl.BlockSpec((1, SEQ, 1), lambda b, h: (b * N_HASHES + h, 0, 0)),
            pl.BlockSpec((1, SEQ, 1), lambda b, h: (b * N_HASHES + h, 0, 0)),
        ],
        out_shape=[
            jax.ShapeDtypeStruct((TASKS, SEQ, 1), jnp.int32),
            jax.ShapeDtypeStruct((TASKS, SEQ, 1), jnp.int32),
        ],
    )(qk, rot_t)


def _permute_body(qk_hbm, v_hbm, pos_hbm, bkt_hbm,
                  sqk_hbm, sv_hbm, st_hbm, sbk_hbm,
                  pos_v, bkt_v, st_v, sbk_v, idxg_v, bufq, bufv, sem):
    nc = 2
    wid = lax.axis_index("s") * nc + lax.axis_index("c")

    def one_task(r, _):
        t = wid * 2 + r
        b = t // N_HASHES
        toff = pl.multiple_of(t * SEQ, SEQ)
        pltpu.sync_copy(pos_hbm.at[pl.ds(toff, SEQ)], pos_v)
        pltpu.sync_copy(bkt_hbm.at[pl.ds(toff, SEQ)], bkt_v)

        boff = b * SEQ

        def scat(j, _):
            base = pl.multiple_of(j * 16, 16)
            idx = pos_v[pl.ds(base, 16)]
            tok = lax.iota(jnp.int32, 16) + base
            plsc.store_scatter(st_v, [idx], tok)
            plsc.store_scatter(idxg_v, [idx], tok + boff)
            plsc.store_scatter(sbk_v, [idx], bkt_v[pl.ds(base, 16)])
            return 0

        lax.fori_loop(0, SEQ // 16, scat, 0)

        pltpu.sync_copy(st_v, st_hbm.at[pl.ds(toff, SEQ)])
        pltpu.sync_copy(sbk_v, sbk_hbm.at[pl.ds(toff, SEQ)])

        def gat(c, _):
            goff = pl.multiple_of(c * GCHUNK, GCHUNK)
            idxs = idxg_v.at[pl.ds(goff, GCHUNK)]
            pltpu.async_copy(qk_hbm.at[idxs], bufq, sem).wait()
            pltpu.sync_copy(bufq, sqk_hbm.at[pl.ds(toff + goff, GCHUNK)])
            pltpu.async_copy(v_hbm.at[idxs], bufv, sem).wait()
            pltpu.sync_copy(bufv, sv_hbm.at[pl.ds(toff + goff, GCHUNK)])
            return 0

        lax.fori_loop(0, SEQ // GCHUNK, gat, 0)
        return 0

    lax.fori_loop(0, TASKS // 32, one_task, 0)


def _permute(qk_flat, v_flat, pos_flat, bkt_flat):
    tot = TASKS * SEQ
    mesh = plsc.VectorSubcoreMesh(core_axis_name="c", subcore_axis_name="s")
    fn = functools.partial(
        pl.kernel,
        out_type=[
            jax.ShapeDtypeStruct((tot, DIM), jnp.float32),
            jax.ShapeDtypeStruct((tot, DIM), jnp.float32),
            jax.ShapeDtypeStruct((tot,), jnp.int32),
            jax.ShapeDtypeStruct((tot,), jnp.int32),
        ],
        mesh=mesh,
        compiler_params=pltpu.CompilerParams(
            needs_layout_passes=False, use_tc_tiling_on_sc=False),
        scratch_types=[
            pltpu.VMEM((SEQ,), jnp.int32),
            pltpu.VMEM((SEQ,), jnp.int32),
            pltpu.VMEM((SEQ,), jnp.int32),
            pltpu.VMEM((SEQ,), jnp.int32),
            pltpu.VMEM((SEQ,), jnp.int32),
            pltpu.VMEM((GCHUNK, DIM), jnp.float32),
            pltpu.VMEM((GCHUNK, DIM), jnp.float32),
            pltpu.SemaphoreType.DMA,
        ],
    )(_permute_body)
    return fn(qk_flat, v_flat, pos_flat, bkt_flat)


def _attend_body(sqk_ref, sv_ref, stq_ref, stk_ref, sbq_ref, sbkk_ref,
                 so_ref, slse_ref):
    sqk = sqk_ref[0]
    sv = sv_ref[0]
    stq = stq_ref[0].reshape(N_CHUNKS, BUCKET_SIZE, 1)
    sbq = sbq_ref[0].reshape(N_CHUNKS, BUCKET_SIZE, 1)
    tk = stk_ref[0]
    bkk = sbkk_ref[0]

    norms = jnp.sqrt(jnp.sum(sqk * sqk, axis=-1, keepdims=True))
    kn = sqk / jnp.maximum(norms, 1e-12)

    q = sqk.reshape(N_CHUNKS, BUCKET_SIZE, DIM)
    kc = kn.reshape(N_CHUNKS, BUCKET_SIZE, DIM)
    kwin = jnp.concatenate(
        [kc, jnp.concatenate([kc[-1:], kc[:-1]], axis=0)], axis=1)
    vc = sv.reshape(N_CHUNKS, BUCKET_SIZE, DIM)
    vwin = jnp.concatenate(
        [vc, jnp.concatenate([vc[-1:], vc[:-1]], axis=0)], axis=1)
    twin = jnp.concatenate(
        [tk, jnp.concatenate([tk[-1:], tk[:-1]], axis=0)],
        axis=1).reshape(N_CHUNKS, 1, 2 * BUCKET_SIZE)
    bwin = jnp.concatenate(
        [bkk, jnp.concatenate([jnp.full((1, BUCKET_SIZE), -1, jnp.int32),
                               bkk[:-1]], axis=0)],
        axis=1).reshape(N_CHUNKS, 1, 2 * BUCKET_SIZE)

    dots = lax.dot_general(q, kwin, (((2,), (2,)), ((0,), (0,))))
    dots = dots * (DIM ** -0.5)
    dots = jnp.where(stq == twin, NEG_SELF, dots)
    dots = jnp.where(sbq != bwin, -jnp.finfo(jnp.float32).max, dots)

    m = jnp.max(dots, axis=-1, keepdims=True)
    e = jnp.exp(dots - m)
    s = jnp.sum(e, axis=-1, keepdims=True)
    lse = m + jnp.log(s)
    p = e * (1.0 / s)
    bo = lax.dot_general(p, vwin, (((2,), (1,)), ((0,), (0,))))
    so_ref[0] = bo.reshape(SEQ, DIM)
    slse_ref[0] = lse.reshape(SEQ, 1)


def _attend(sqk, sv, stq, stk, sbq, sbkk):
    return pl.pallas_call(
        _attend_body,
        grid=(TASKS,),
        in_specs=[
            pl.BlockSpec((1, SEQ, DIM), lambda t: (t, 0, 0)),
            pl.BlockSpec((1, SEQ, DIM), lambda t: (t, 0, 0)),
            pl.BlockSpec((1, SEQ, 1), lambda t: (t, 0, 0)),
            pl.BlockSpec((1, N_CHUNKS, BUCKET_SIZE), lambda t: (t, 0, 0)),
            pl.BlockSpec((1, SEQ, 1), lambda t: (t, 0, 0)),
            pl.BlockSpec((1, N_CHUNKS, BUCKET_SIZE), lambda t: (t, 0, 0)),
        ],
        out_specs=[
            pl.BlockSpec((1, SEQ, DIM), lambda t: (t, 0, 0)),
            pl.BlockSpec((1, SEQ, 1), lambda t: (t, 0, 0)),
        ],
        out_shape=[
            jax.ShapeDtypeStruct((TASKS, SEQ, DIM), jnp.float32),
            jax.ShapeDtypeStruct((TASKS, SEQ, 1), jnp.float32),
        ],
    )(sqk, sv, stq, stk, sbq, sbkk)


def _unpermute_body(so_hbm, slse_hbm, pos_hbm,
                    o_hbm, lg_hbm,
                    pos_v, gidx_v, lse_v, lgo_v, bufo, sem):
    nc = 2
    wid = lax.axis_index("s") * nc + lax.axis_index("c")

    def one_task(r, _):
        t = wid * 2 + r
        toff = pl.multiple_of(t * SEQ, SEQ)
        pltpu.sync_copy(pos_hbm.at[pl.ds(toff, SEQ)], pos_v)
        pltpu.sync_copy(slse_hbm.at[pl.ds(toff, SEQ)], lse_v)

        def addoff(j, _):
            base = pl.multiple_of(j * 16, 16)
            idx = pos_v[pl.ds(base, 16)]
            gidx_v[pl.ds(base, 16)] = idx + toff
            lgo_v[pl.ds(base, 16)] = plsc.load_gather(lse_v, [idx])
            return 0

        lax.fori_loop(0, SEQ // 16, addoff, 0)
        pltpu.sync_copy(lgo_v, lg_hbm.at[pl.ds(toff, SEQ)])

        def gat(c, _):
            goff = pl.multiple_of(c * GCHUNK, GCHUNK)
            idxs = gidx_v.at[pl.ds(goff, GCHUNK)]
            pltpu.async_copy(so_hbm.at[idxs], bufo, sem).wait()
            pltpu.sync_copy(bufo, o_hbm.at[pl.ds(toff + goff, GCHUNK)])
            return 0

        lax.fori_loop(0, SEQ // GCHUNK, gat, 0)
        return 0

    lax.fori_loop(0, TASKS // 32, one_task, 0)


def _unpermute(so_flat, slse_flat, pos_flat):
    tot = TASKS * SEQ
    mesh = plsc.VectorSubcoreMesh(core_axis_name="c", subcore_axis_name="s")
    fn = functools.partial(
        pl.kernel,
        out_type=[
            jax.ShapeDtypeStruct((tot, DIM), jnp.float32),
            jax.ShapeDtypeStruct((tot,), jnp.float32),
        ],
        mesh=mesh,
        compiler_params=pltpu.CompilerParams(
            needs_layout_passes=False, use_tc_tiling_on_sc=False),
        scratch_types=[
            pltpu.VMEM((SEQ,), jnp.int32),
            pltpu.VMEM((SEQ,), jnp.int32),
            pltpu.VMEM((SEQ,), jnp.float32),
            pltpu.VMEM((SEQ,), jnp.float32),
            pltpu.VMEM((GCHUNK, DIM), jnp.float32),
            pltpu.SemaphoreType.DMA,
        ],
    )(_unpermute_body)
    return fn(so_flat, slse_flat, pos_flat)


def _combine_body(o_ref, lg_ref, out_ref):
    o = o_ref[0]
    lg = lg_ref[0]
    m = jnp.max(lg, axis=0, keepdims=True)
    s = jnp.sum(jnp.exp(lg - m), axis=0, keepdims=True)
    lse = m + jnp.log(s)
    p = jnp.exp(lg - lse)
    out_ref[0] = jnp.sum(o * p, axis=0)


_CSEQ = 512


def _combine(o4, lg4):
    return pl.pallas_call(
        _combine_body,
        grid=(BATCH, SEQ // _CSEQ),
        in_specs=[
            pl.BlockSpec((1, N_HASHES, _CSEQ, DIM), lambda b, s: (b, 0, s, 0)),
            pl.BlockSpec((1, N_HASHES, _CSEQ, 1), lambda b, s: (b, 0, s, 0)),
        ],
        out_specs=pl.BlockSpec((1, _CSEQ, DIM), lambda b, s: (b, s, 0)),
        out_shape=jax.ShapeDtypeStruct((BATCH, SEQ, DIM), jnp.float32),
    )(o4, lg4)


def kernel(qk, v):
    rot = jax.random.normal(jax.random.key(42),
                            (DIM, N_HASHES, N_BUCKETS // 2), dtype=qk.dtype)
    rot_t = jnp.transpose(rot, (1, 0, 2))

    bucket, pos = _hash_sort(qk, rot_t)
    pos_flat = pos.reshape(TASKS * SEQ)
    bkt_flat = bucket.reshape(TASKS * SEQ)

    qk_flat = qk.reshape(BATCH * SEQ, DIM)
    v_flat = v.reshape(BATCH * SEQ, DIM)
    sqk_flat, sv_flat, st_flat, sbk_flat = _permute(
        qk_flat, v_flat, pos_flat, bkt_flat)

    so, slse = _attend(sqk_flat.reshape(TASKS, SEQ, DIM),
                       sv_flat.reshape(TASKS, SEQ, DIM),
                       st_flat.reshape(TASKS, SEQ, 1),
                       st_flat.reshape(TASKS, N_CHUNKS, BUCKET_SIZE),
                       sbk_flat.reshape(TASKS, SEQ, 1),
                       sbk_flat.reshape(TASKS, N_CHUNKS, BUCKET_SIZE))

    o_flat, lg_flat = _unpermute(so.reshape(TASKS * SEQ, DIM),
                                 slse.reshape(TASKS * SEQ),
                                 pos_flat)

    out = _combine(o_flat.reshape(BATCH, N_HASHES, SEQ, DIM),
                   lg_flat.reshape(BATCH, N_HASHES, SEQ, 1))
    return out


# speedup vs baseline: 8.1588x; 1.4505x over previous
import functools

import jax
import jax.numpy as jnp
from jax import lax
from jax.experimental import pallas as pl
from jax.experimental.pallas import tpu as pltpu
from jax.experimental.pallas import tpu_sc as plsc

BATCH = 8
SEQ = 4096
DIM = 64
N_HASHES = 8
BUCKET_SIZE = 64
N_BUCKETS = SEQ // BUCKET_SIZE
N_CHUNKS = SEQ // BUCKET_SIZE
TASKS = BATCH * N_HASHES
NEG_SELF = -10000.0
GCHUNK = 128


def _hash_sort_body(qk_ref, rot_ref, pos_ref, off_ref, cnt_ref):
    qk = qk_ref[0]
    rot = rot_ref[0]
    r = lax.dot_general(qk, rot, (((1,), (0,)), ((), ())),
                        preferred_element_type=jnp.float32)
    r2 = jnp.concatenate([r, -r], axis=-1)
    m = jnp.max(r2, axis=-1, keepdims=True)
    col = lax.broadcasted_iota(jnp.int32, r2.shape, 1)
    bucket = jnp.min(jnp.where(r2 == m, col, N_BUCKETS), axis=-1,
                     keepdims=True)

    onehot = (bucket == lax.broadcasted_iota(
        jnp.int32, (SEQ, N_BUCKETS), 1)).astype(jnp.float32)

    cum = onehot
    k = 1
    while k < SEQ:
        cum = cum + jnp.concatenate(
            [jnp.zeros((k, N_BUCKETS), jnp.float32), cum[:-k]], axis=0)
        k *= 2

    counts = jnp.sum(onehot, axis=0, keepdims=True)
    ci = counts.astype(jnp.int32)
    hi = (ci >> 8).astype(jnp.float32)
    lo = (ci & 255).astype(jnp.float32)
    tri = (lax.broadcasted_iota(jnp.int32, (N_BUCKETS, N_BUCKETS), 0)
           < lax.broadcasted_iota(
               jnp.int32, (N_BUCKETS, N_BUCKETS), 1)).astype(jnp.float32)
    off = (lax.dot_general(hi, tri, (((1,), (0,)), ((), ()))) * 256.0
           + lax.dot_general(lo, tri, (((1,), (0,)), ((), ()))))

    pos = jnp.sum(onehot * (cum - 1.0 + off), axis=-1, keepdims=True)
    pos_ref[0] = pos.astype(jnp.int32).reshape(SEQ // 128, 128)
    off_ref[0] = off.astype(jnp.int32)
    cnt_ref[0] = ci


def _hash_sort(qk, rot_t):
    return pl.pallas_call(
        _hash_sort_body,
        grid=(BATCH, N_HASHES),
        in_specs=[
            pl.BlockSpec((1, SEQ, DIM), lambda b, h: (b, 0, 0)),
            pl.BlockSpec((1, DIM, N_BUCKETS // 2), lambda b, h: (h, 0, 0)),
        ],
        out_specs=[
            pl.BlockSpec((1, SEQ // 128, 128),
                         lambda b, h: (b * N_HASHES + h, 0, 0)),
            pl.BlockSpec((1, 1, N_BUCKETS),
                         lambda b, h: (b * N_HASHES + h, 0, 0)),
            pl.BlockSpec((1, 1, N_BUCKETS),
                         lambda b, h: (b * N_HASHES + h, 0, 0)),
        ],
        out_shape=[
            jax.ShapeDtypeStruct((TASKS, SEQ // 128, 128), jnp.int32),
            jax.ShapeDtypeStruct((TASKS, 1, N_BUCKETS), jnp.int32),
            jax.ShapeDtypeStruct((TASKS, 1, N_BUCKETS), jnp.int32),
        ],
    )(qk, rot_t)


def _permute_body(qk_hbm, v_hbm, pos_hbm,
                  sqk_hbm, sv_hbm,
                  pos_v, idxg_v, bufq, bufv, sem):
    nc = 2
    wid = lax.axis_index("s") * nc + lax.axis_index("c")

    def one_task(r, _):
        t = wid * 2 + r
        b = t // N_HASHES
        toff = pl.multiple_of(t * SEQ, SEQ)
        pltpu.sync_copy(pos_hbm.at[pl.ds(toff, SEQ)], pos_v)

        boff = b * SEQ

        def scat(j, _):
            base = pl.multiple_of(j * 16, 16)
            idx = pos_v[pl.ds(base, 16)]
            tok = lax.iota(jnp.int32, 16) + base
            plsc.store_scatter(idxg_v, [idx], tok + boff)
            return 0

        lax.fori_loop(0, SEQ // 16, scat, 0)

        def gat(c, _):
            goff = pl.multiple_of(c * GCHUNK, GCHUNK)
            idxs = idxg_v.at[pl.ds(goff, GCHUNK)]
            pltpu.async_copy(qk_hbm.at[idxs], bufq, sem).wait()
            pltpu.sync_copy(bufq, sqk_hbm.at[pl.ds(toff + goff, GCHUNK)])
            pltpu.async_copy(v_hbm.at[idxs], bufv, sem).wait()
            pltpu.sync_copy(bufv, sv_hbm.at[pl.ds(toff + goff, GCHUNK)])
            return 0

        lax.fori_loop(0, SEQ // GCHUNK, gat, 0)
        return 0

    lax.fori_loop(0, TASKS // 32, one_task, 0)


def _permute(qk_flat, v_flat, pos_flat):
    tot = TASKS * SEQ
    mesh = plsc.VectorSubcoreMesh(core_axis_name="c", subcore_axis_name="s")
    fn = functools.partial(
        pl.kernel,
        out_type=[
            jax.ShapeDtypeStruct((tot, DIM), jnp.float32),
            jax.ShapeDtypeStruct((tot, DIM), jnp.float32),
        ],
        mesh=mesh,
        compiler_params=pltpu.CompilerParams(
            needs_layout_passes=False, use_tc_tiling_on_sc=False),
        scratch_types=[
            pltpu.VMEM((SEQ,), jnp.int32),
            pltpu.VMEM((SEQ,), jnp.int32),
            pltpu.VMEM((GCHUNK, DIM), jnp.float32),
            pltpu.VMEM((GCHUNK, DIM), jnp.float32),
            pltpu.SemaphoreType.DMA,
        ],
    )(_permute_body)
    return fn(qk_flat, v_flat, pos_flat)


def _attend_body(sqk_ref, sv_ref, off_ref, cnt_ref, so_ref, slse_ref):
    sqk = sqk_ref[0]
    sv = sv_ref[0]
    off = off_ref[0]
    cnt = cnt_ref[0]

    norms = jnp.sqrt(jnp.sum(sqk * sqk, axis=-1, keepdims=True))
    kn = sqk / jnp.maximum(norms, 1e-12)

    q = sqk.reshape(N_CHUNKS, BUCKET_SIZE, DIM)
    kc = kn.reshape(N_CHUNKS, BUCKET_SIZE, DIM)
    kwin = jnp.concatenate(
        [kc, jnp.concatenate([kc[-1:], kc[:-1]], axis=0)], axis=1)
    vc = sv.reshape(N_CHUNKS, BUCKET_SIZE, DIM)
    vwin = jnp.concatenate(
        [vc, jnp.concatenate([vc[-1:], vc[:-1]], axis=0)], axis=1)

    ji = lax.broadcasted_iota(jnp.int32, (SEQ, N_BUCKETS), 0)
    ohs = ((ji >= off) & (ji < off + cnt)).astype(jnp.float32)
    oq = ohs.reshape(N_CHUNKS, BUCKET_SIZE, N_BUCKETS)
    okw = jnp.concatenate(
        [oq, jnp.concatenate(
            [jnp.zeros((1, BUCKET_SIZE, N_BUCKETS), jnp.float32), oq[:-1]],
            axis=0)], axis=1)
    same = lax.dot_general(oq, okw, (((2,), (2,)), ((0,), (0,))))

    dots = lax.dot_general(q, kwin, (((2,), (2,)), ((0,), (0,))))
    dots = dots * (DIM ** -0.5)
    qi = lax.broadcasted_iota(jnp.int32, dots.shape, 1)
    zi = lax.broadcasted_iota(jnp.int32, dots.shape, 2)
    dots = jnp.where(qi == zi, NEG_SELF, dots)
    dots = jnp.where(same < 0.5, -jnp.finfo(jnp.float32).max, dots)

    m = jnp.max(dots, axis=-1, keepdims=True)
    e = jnp.exp(dots - m)
    s = jnp.sum(e, axis=-1, keepdims=True)
    lse = m + jnp.log(s)
    p = e * (1.0 / s)
    bo = lax.dot_general(p, vwin, (((2,), (1,)), ((0,), (0,))))
    so_ref[0] = bo.reshape(SEQ, DIM)
    slse_ref[0] = lse.reshape(SEQ, 1).reshape(SEQ // 128, 128)


def _attend(sqk, sv, offs, cnts):
    return pl.pallas_call(
        _attend_body,
        grid=(TASKS,),
        in_specs=[
            pl.BlockSpec((1, SEQ, DIM), lambda t: (t, 0, 0)),
            pl.BlockSpec((1, SEQ, DIM), lambda t: (t, 0, 0)),
            pl.BlockSpec((1, 1, N_BUCKETS), lambda t: (t, 0, 0)),
            pl.BlockSpec((1, 1, N_BUCKETS), lambda t: (t, 0, 0)),
        ],
        out_specs=[
            pl.BlockSpec((1, SEQ, DIM), lambda t: (t, 0, 0)),
            pl.BlockSpec((1, SEQ // 128, 128), lambda t: (t, 0, 0)),
        ],
        out_shape=[
            jax.ShapeDtypeStruct((TASKS, SEQ, DIM), jnp.float32),
            jax.ShapeDtypeStruct((TASKS, SEQ // 128, 128), jnp.float32),
        ],
    )(sqk, sv, offs, cnts)


def _unpermute_body(so_hbm, slse_hbm, pos_hbm,
                    o_hbm, lg_hbm,
                    pos_v, gidx_v, lse_v, lgo_v, bufo, sem):
    nc = 2
    wid = lax.axis_index("s") * nc + lax.axis_index("c")

    def one_task(r, _):
        t = wid * 2 + r
        toff = pl.multiple_of(t * SEQ, SEQ)
        pltpu.sync_copy(pos_hbm.at[pl.ds(toff, SEQ)], pos_v)
        pltpu.sync_copy(slse_hbm.at[pl.ds(toff, SEQ)], lse_v)

        def addoff(j, _):
            base = pl.multiple_of(j * 16, 16)
            idx = pos_v[pl.ds(base, 16)]
            gidx_v[pl.ds(base, 16)] = idx + toff
            lgo_v[pl.ds(base, 16)] = plsc.load_gather(lse_v, [idx])
            return 0

        lax.fori_loop(0, SEQ // 16, addoff, 0)
        pltpu.sync_copy(lgo_v, lg_hbm.at[pl.ds(toff, SEQ)])

        def gat(c, _):
            goff = pl.multiple_of(c * GCHUNK, GCHUNK)
            idxs = gidx_v.at[pl.ds(goff, GCHUNK)]
            pltpu.async_copy(so_hbm.at[idxs], bufo, sem).wait()
            pltpu.sync_copy(bufo, o_hbm.at[pl.ds(toff + goff, GCHUNK)])
            return 0

        lax.fori_loop(0, SEQ // GCHUNK, gat, 0)
        return 0

    lax.fori_loop(0, TASKS // 32, one_task, 0)


def _unpermute(so_flat, slse_flat, pos_flat):
    tot = TASKS * SEQ
    mesh = plsc.VectorSubcoreMesh(core_axis_name="c", subcore_axis_name="s")
    fn = functools.partial(
        pl.kernel,
        out_type=[
            jax.ShapeDtypeStruct((tot, DIM), jnp.float32),
            jax.ShapeDtypeStruct((tot,), jnp.float32),
        ],
        mesh=mesh,
        compiler_params=pltpu.CompilerParams(
            needs_layout_passes=False, use_tc_tiling_on_sc=False),
        scratch_types=[
            pltpu.VMEM((SEQ,), jnp.int32),
            pltpu.VMEM((SEQ,), jnp.int32),
            pltpu.VMEM((SEQ,), jnp.float32),
            pltpu.VMEM((SEQ,), jnp.float32),
            pltpu.VMEM((GCHUNK, DIM), jnp.float32),
            pltpu.SemaphoreType.DMA,
        ],
    )(_unpermute_body)
    return fn(so_flat, slse_flat, pos_flat)


_CSEQ = 512


def _combine_body(o_ref, lg_ref, out_ref):
    o = o_ref[0]
    lg = lg_ref[0]
    m = jnp.max(lg, axis=-1, keepdims=True)
    e = jnp.exp(lg - m)
    s = jnp.sum(e, axis=-1, keepdims=True)
    p = e / s
    acc = o[0] * p[:, 0:1]
    for h in range(1, N_HASHES):
        acc = acc + o[h] * p[:, h:h + 1]
    out_ref[0] = acc


def _combine(o4, lg3t):
    return pl.pallas_call(
        _combine_body,
        grid=(BATCH, SEQ // _CSEQ),
        in_specs=[
            pl.BlockSpec((1, N_HASHES, _CSEQ, DIM), lambda b, s: (b, 0, s, 0)),
            pl.BlockSpec((1, _CSEQ, N_HASHES), lambda b, s: (b, s, 0)),
        ],
        out_specs=pl.BlockSpec((1, _CSEQ, DIM), lambda b, s: (b, s, 0)),
        out_shape=jax.ShapeDtypeStruct((BATCH, SEQ, DIM), jnp.float32),
    )(o4, lg3t)


def kernel(qk, v):
    rot = jax.random.normal(jax.random.key(42),
                            (DIM, N_HASHES, N_BUCKETS // 2), dtype=qk.dtype)
    rot_t = jnp.transpose(rot, (1, 0, 2))

    pos, offs, cnts = _hash_sort(qk, rot_t)
    pos_flat = pos.reshape(TASKS * SEQ)

    qk_flat = qk.reshape(BATCH * SEQ, DIM)
    v_flat = v.reshape(BATCH * SEQ, DIM)
    sqk_flat, sv_flat = _permute(qk_flat, v_flat, pos_flat)

    so, slse = _attend(sqk_flat.reshape(TASKS, SEQ, DIM),
                       sv_flat.reshape(TASKS, SEQ, DIM),
                       offs, cnts)

    o_flat, lg_flat = _unpermute(so.reshape(TASKS * SEQ, DIM),
                                 slse.reshape(TASKS * SEQ),
                                 pos_flat)

    lg3t = jnp.transpose(lg_flat.reshape(BATCH, N_HASHES, SEQ), (0, 2, 1))
    out = _combine(o_flat.reshape(BATCH, N_HASHES, SEQ, DIM), lg3t)
    return out


# speedup vs baseline: 8.9613x; 1.0984x over previous
import functools

import jax
import jax.numpy as jnp
from jax import lax
from jax.experimental import pallas as pl
from jax.experimental.pallas import tpu as pltpu
from jax.experimental.pallas import tpu_sc as plsc

BATCH = 8
SEQ = 4096
DIM = 64
N_HASHES = 8
BUCKET_SIZE = 64
N_BUCKETS = SEQ // BUCKET_SIZE
N_CHUNKS = SEQ // BUCKET_SIZE
TASKS = BATCH * N_HASHES
NEG_SELF = -10000.0
GCHUNK = 128


def _hash_sort_body(qk_ref, rot_ref, pos_ref, off_ref, cnt_ref):
    qk = qk_ref[0]
    rot = rot_ref[0]
    r = lax.dot_general(qk, rot, (((1,), (0,)), ((), ())),
                        preferred_element_type=jnp.float32)
    r2 = jnp.concatenate([r, -r], axis=-1)
    m = jnp.max(r2, axis=-1, keepdims=True)
    col = lax.broadcasted_iota(jnp.int32, r2.shape, 1)
    bucket = jnp.min(jnp.where(r2 == m, col, N_BUCKETS), axis=-1,
                     keepdims=True)

    onehot = (bucket == lax.broadcasted_iota(
        jnp.int32, (SEQ, N_BUCKETS), 1)).astype(jnp.float32)

    cum = onehot
    k = 1
    while k < SEQ:
        cum = cum + jnp.concatenate(
            [jnp.zeros((k, N_BUCKETS), jnp.float32), cum[:-k]], axis=0)
        k *= 2

    counts = jnp.sum(onehot, axis=0, keepdims=True)
    ci = counts.astype(jnp.int32)
    hi = (ci >> 8).astype(jnp.float32)
    lo = (ci & 255).astype(jnp.float32)
    tri = (lax.broadcasted_iota(jnp.int32, (N_BUCKETS, N_BUCKETS), 0)
           < lax.broadcasted_iota(
               jnp.int32, (N_BUCKETS, N_BUCKETS), 1)).astype(jnp.float32)
    off = (lax.dot_general(hi, tri, (((1,), (0,)), ((), ()))) * 256.0
           + lax.dot_general(lo, tri, (((1,), (0,)), ((), ()))))

    pos = jnp.sum(onehot * (cum - 1.0 + off), axis=-1, keepdims=True)
    pos_ref[0] = pos.astype(jnp.int32).reshape(SEQ // 128, 128)
    off_ref[0] = off.astype(jnp.int32)
    cnt_ref[0] = ci


def _hash_sort(qk, rot_t):
    return pl.pallas_call(
        _hash_sort_body,
        grid=(BATCH, N_HASHES),
        in_specs=[
            pl.BlockSpec((1, SEQ, DIM), lambda b, h: (b, 0, 0)),
            pl.BlockSpec((1, DIM, N_BUCKETS // 2), lambda b, h: (h, 0, 0)),
        ],
        out_specs=[
            pl.BlockSpec((1, SEQ // 128, 128),
                         lambda b, h: (b * N_HASHES + h, 0, 0)),
            pl.BlockSpec((1, 1, N_BUCKETS),
                         lambda b, h: (b * N_HASHES + h, 0, 0)),
            pl.BlockSpec((1, 1, N_BUCKETS),
                         lambda b, h: (b * N_HASHES + h, 0, 0)),
        ],
        out_shape=[
            jax.ShapeDtypeStruct((TASKS, SEQ // 128, 128), jnp.int32),
            jax.ShapeDtypeStruct((TASKS, 1, N_BUCKETS), jnp.int32),
            jax.ShapeDtypeStruct((TASKS, 1, N_BUCKETS), jnp.int32),
        ],
    )(qk, rot_t)


def _permute_body(qk_hbm, v_hbm, pos_hbm,
                  sqk_hbm, sv_hbm,
                  pos_v, idxg_v, bufq, bufv, sem):
    nc = 2
    wid = lax.axis_index("s") * nc + lax.axis_index("c")

    def one_task(r, _):
        t = wid * 2 + r
        b = t // N_HASHES
        toff = pl.multiple_of(t * SEQ, SEQ)
        pltpu.sync_copy(pos_hbm.at[pl.ds(toff, SEQ)], pos_v)

        boff = b * SEQ

        def scat(j, _):
            base = pl.multiple_of(j * 16, 16)
            idx = pos_v[pl.ds(base, 16)]
            tok = lax.iota(jnp.int32, 16) + base
            plsc.store_scatter(idxg_v, [idx], tok + boff)
            return 0

        lax.fori_loop(0, SEQ // 16, scat, 0)

        def gat(g, _):
            goff = pl.multiple_of(g * (4 * GCHUNK), 4 * GCHUNK)
            waits = []
            for j in range(4):
                idxs = idxg_v.at[pl.ds(goff + j * GCHUNK, GCHUNK)]
                dq = bufq.at[pl.ds(j * GCHUNK, GCHUNK)]
                dv = bufv.at[pl.ds(j * GCHUNK, GCHUNK)]
                waits.append(pltpu.async_copy(qk_hbm.at[idxs], dq, sem))
                waits.append(pltpu.async_copy(v_hbm.at[idxs], dv, sem))
            for w in waits:
                w.wait()
            pltpu.sync_copy(bufq, sqk_hbm.at[pl.ds(toff + goff, 4 * GCHUNK)])
            pltpu.sync_copy(bufv, sv_hbm.at[pl.ds(toff + goff, 4 * GCHUNK)])
            return 0

        lax.fori_loop(0, SEQ // (4 * GCHUNK), gat, 0)
        return 0

    lax.fori_loop(0, TASKS // 32, one_task, 0)


def _permute(qk_flat, v_flat, pos_flat):
    tot = TASKS * SEQ
    mesh = plsc.VectorSubcoreMesh(core_axis_name="c", subcore_axis_name="s")
    fn = functools.partial(
        pl.kernel,
        out_type=[
            jax.ShapeDtypeStruct((tot, DIM), jnp.float32),
            jax.ShapeDtypeStruct((tot, DIM), jnp.float32),
        ],
        mesh=mesh,
        compiler_params=pltpu.CompilerParams(
            needs_layout_passes=False, use_tc_tiling_on_sc=False),
        scratch_types=[
            pltpu.VMEM((SEQ,), jnp.int32),
            pltpu.VMEM((SEQ,), jnp.int32),
            pltpu.VMEM((4 * GCHUNK, DIM), jnp.float32),
            pltpu.VMEM((4 * GCHUNK, DIM), jnp.float32),
            pltpu.SemaphoreType.DMA,
        ],
    )(_permute_body)
    return fn(qk_flat, v_flat, pos_flat)


def _attend_body(sqk_ref, sv_ref, off_ref, cnt_ref, so_ref, slse_ref):
    sqk = sqk_ref[0]
    sv = sv_ref[0]
    off = off_ref[0]
    cnt = cnt_ref[0]

    norms = jnp.sqrt(jnp.sum(sqk * sqk, axis=-1, keepdims=True))
    kn = sqk / jnp.maximum(norms, 1e-12)

    q = sqk.reshape(N_CHUNKS, BUCKET_SIZE, DIM)
    kc = kn.reshape(N_CHUNKS, BUCKET_SIZE, DIM)
    kwin = jnp.concatenate(
        [kc, jnp.concatenate([kc[-1:], kc[:-1]], axis=0)], axis=1)
    vc = sv.reshape(N_CHUNKS, BUCKET_SIZE, DIM)
    vwin = jnp.concatenate(
        [vc, jnp.concatenate([vc[-1:], vc[:-1]], axis=0)], axis=1)

    ji = lax.broadcasted_iota(jnp.int32, (SEQ, N_BUCKETS), 0)
    ohs = ((ji >= off) & (ji < off + cnt)).astype(jnp.float32)
    oq = ohs.reshape(N_CHUNKS, BUCKET_SIZE, N_BUCKETS)
    okw = jnp.concatenate(
        [oq, jnp.concatenate(
            [jnp.zeros((1, BUCKET_SIZE, N_BUCKETS), jnp.float32), oq[:-1]],
            axis=0)], axis=1)
    same = lax.dot_general(oq, okw, (((2,), (2,)), ((0,), (0,))))

    dots = lax.dot_general(q, kwin, (((2,), (2,)), ((0,), (0,))))
    dots = dots * (DIM ** -0.5)
    qi = lax.broadcasted_iota(jnp.int32, dots.shape, 1)
    zi = lax.broadcasted_iota(jnp.int32, dots.shape, 2)
    dots = jnp.where(qi == zi, NEG_SELF, dots)
    dots = jnp.where(same < 0.5, -jnp.finfo(jnp.float32).max, dots)

    m = jnp.max(dots, axis=-1, keepdims=True)
    e = jnp.exp(dots - m)
    s = jnp.sum(e, axis=-1, keepdims=True)
    lse = m + jnp.log(s)
    p = e * (1.0 / s)
    bo = lax.dot_general(p, vwin, (((2,), (1,)), ((0,), (0,))))
    so_ref[0] = bo.reshape(SEQ, DIM)
    slse_ref[0] = lse.reshape(SEQ, 1).reshape(SEQ // 128, 128)


def _attend(sqk, sv, offs, cnts):
    return pl.pallas_call(
        _attend_body,
        grid=(TASKS,),
        in_specs=[
            pl.BlockSpec((1, SEQ, DIM), lambda t: (t, 0, 0)),
            pl.BlockSpec((1, SEQ, DIM), lambda t: (t, 0, 0)),
            pl.BlockSpec((1, 1, N_BUCKETS), lambda t: (t, 0, 0)),
            pl.BlockSpec((1, 1, N_BUCKETS), lambda t: (t, 0, 0)),
        ],
        out_specs=[
            pl.BlockSpec((1, SEQ, DIM), lambda t: (t, 0, 0)),
            pl.BlockSpec((1, SEQ // 128, 128), lambda t: (t, 0, 0)),
        ],
        out_shape=[
            jax.ShapeDtypeStruct((TASKS, SEQ, DIM), jnp.float32),
            jax.ShapeDtypeStruct((TASKS, SEQ // 128, 128), jnp.float32),
        ],
    )(sqk, sv, offs, cnts)


def _unpermute_body(so_hbm, slse_hbm, pos_hbm,
                    o_hbm, lg_hbm,
                    pos_v, gidx_v, lse_v, lgo_v, bufo, sem):
    nc = 2
    wid = lax.axis_index("s") * nc + lax.axis_index("c")

    def one_task(r, _):
        t = wid * 2 + r
        toff = pl.multiple_of(t * SEQ, SEQ)
        pltpu.sync_copy(pos_hbm.at[pl.ds(toff, SEQ)], pos_v)
        pltpu.sync_copy(slse_hbm.at[pl.ds(toff, SEQ)], lse_v)

        def addoff(j, _):
            base = pl.multiple_of(j * 16, 16)
            idx = pos_v[pl.ds(base, 16)]
            gidx_v[pl.ds(base, 16)] = idx + toff
            lgo_v[pl.ds(base, 16)] = plsc.load_gather(lse_v, [idx])
            return 0

        lax.fori_loop(0, SEQ // 16, addoff, 0)
        pltpu.sync_copy(lgo_v, lg_hbm.at[pl.ds(toff, SEQ)])

        def gat(g, _):
            goff = pl.multiple_of(g * (4 * GCHUNK), 4 * GCHUNK)
            waits = []
            for j in range(4):
                idxs = gidx_v.at[pl.ds(goff + j * GCHUNK, GCHUNK)]
                do = bufo.at[pl.ds(j * GCHUNK, GCHUNK)]
                waits.append(pltpu.async_copy(so_hbm.at[idxs], do, sem))
            for w in waits:
                w.wait()
            pltpu.sync_copy(bufo, o_hbm.at[pl.ds(toff + goff, 4 * GCHUNK)])
            return 0

        lax.fori_loop(0, SEQ // (4 * GCHUNK), gat, 0)
        return 0

    lax.fori_loop(0, TASKS // 32, one_task, 0)


def _unpermute(so_flat, slse_flat, pos_flat):
    tot = TASKS * SEQ
    mesh = plsc.VectorSubcoreMesh(core_axis_name="c", subcore_axis_name="s")
    fn = functools.partial(
        pl.kernel,
        out_type=[
            jax.ShapeDtypeStruct((tot, DIM), jnp.float32),
            jax.ShapeDtypeStruct((tot,), jnp.float32),
        ],
        mesh=mesh,
        compiler_params=pltpu.CompilerParams(
            needs_layout_passes=False, use_tc_tiling_on_sc=False),
        scratch_types=[
            pltpu.VMEM((SEQ,), jnp.int32),
            pltpu.VMEM((SEQ,), jnp.int32),
            pltpu.VMEM((SEQ,), jnp.float32),
            pltpu.VMEM((SEQ,), jnp.float32),
            pltpu.VMEM((4 * GCHUNK, DIM), jnp.float32),
            pltpu.SemaphoreType.DMA,
        ],
    )(_unpermute_body)
    return fn(so_flat, slse_flat, pos_flat)


_CSEQ = 512


def _combine_body(o_ref, lg_ref, out_ref):
    o = o_ref[0]
    lg = lg_ref[0]
    m = jnp.max(lg, axis=-1, keepdims=True)
    e = jnp.exp(lg - m)
    s = jnp.sum(e, axis=-1, keepdims=True)
    p = e / s
    acc = o[0] * p[:, 0:1]
    for h in range(1, N_HASHES):
        acc = acc + o[h] * p[:, h:h + 1]
    out_ref[0] = acc


def _combine(o4, lg3t):
    return pl.pallas_call(
        _combine_body,
        grid=(BATCH, SEQ // _CSEQ),
        in_specs=[
            pl.BlockSpec((1, N_HASHES, _CSEQ, DIM), lambda b, s: (b, 0, s, 0)),
            pl.BlockSpec((1, _CSEQ, N_HASHES), lambda b, s: (b, s, 0)),
        ],
        out_specs=pl.BlockSpec((1, _CSEQ, DIM), lambda b, s: (b, s, 0)),
        out_shape=jax.ShapeDtypeStruct((BATCH, SEQ, DIM), jnp.float32),
    )(o4, lg3t)


def kernel(qk, v):
    rot = jax.random.normal(jax.random.key(42),
                            (DIM, N_HASHES, N_BUCKETS // 2), dtype=qk.dtype)
    rot_t = jnp.transpose(rot, (1, 0, 2))

    pos, offs, cnts = _hash_sort(qk, rot_t)
    pos_flat = pos.reshape(TASKS * SEQ)

    qk_flat = qk.reshape(BATCH * SEQ, DIM)
    v_flat = v.reshape(BATCH * SEQ, DIM)
    sqk_flat, sv_flat = _permute(qk_flat, v_flat, pos_flat)

    so, slse = _attend(sqk_flat.reshape(TASKS, SEQ, DIM),
                       sv_flat.reshape(TASKS, SEQ, DIM),
                       offs, cnts)

    o_flat, lg_flat = _unpermute(so.reshape(TASKS * SEQ, DIM),
                                 slse.reshape(TASKS * SEQ),
                                 pos_flat)

    lg3t = jnp.transpose(lg_flat.reshape(BATCH, N_HASHES, SEQ), (0, 2, 1))
    out = _combine(o_flat.reshape(BATCH, N_HASHES, SEQ, DIM), lg3t)
    return out


# speedup vs baseline: 9.4176x; 1.0509x over previous
import functools

import jax
import jax.numpy as jnp
from jax import lax
from jax.experimental import pallas as pl
from jax.experimental.pallas import tpu as pltpu
from jax.experimental.pallas import tpu_sc as plsc

BATCH = 8
SEQ = 4096
DIM = 64
N_HASHES = 8
BUCKET_SIZE = 64
N_BUCKETS = SEQ // BUCKET_SIZE
N_CHUNKS = SEQ // BUCKET_SIZE
TASKS = BATCH * N_HASHES
NEG_SELF = -10000.0
GCHUNK = 128
NG = 4
GB = BATCH // NG
GT = TASKS // NG


def _hash_sort_body(qk_ref, rot_ref, pos_ref, off_ref, cnt_ref):
    qk = qk_ref[0]
    rot = rot_ref[0]
    r = lax.dot_general(qk, rot, (((1,), (0,)), ((), ())),
                        preferred_element_type=jnp.float32)
    r2 = jnp.concatenate([r, -r], axis=-1)
    m = jnp.max(r2, axis=-1, keepdims=True)
    col = lax.broadcasted_iota(jnp.int32, r2.shape, 1)
    bucket = jnp.min(jnp.where(r2 == m, col, N_BUCKETS), axis=-1,
                     keepdims=True)

    onehot = (bucket == lax.broadcasted_iota(
        jnp.int32, (SEQ, N_BUCKETS), 1)).astype(jnp.float32)

    cum = onehot
    k = 1
    while k < SEQ:
        cum = cum + jnp.concatenate(
            [jnp.zeros((k, N_BUCKETS), jnp.float32), cum[:-k]], axis=0)
        k *= 2

    counts = jnp.sum(onehot, axis=0, keepdims=True)
    ci = counts.astype(jnp.int32)
    hi = (ci >> 8).astype(jnp.float32)
    lo = (ci & 255).astype(jnp.float32)
    tri = (lax.broadcasted_iota(jnp.int32, (N_BUCKETS, N_BUCKETS), 0)
           < lax.broadcasted_iota(
               jnp.int32, (N_BUCKETS, N_BUCKETS), 1)).astype(jnp.float32)
    off = (lax.dot_general(hi, tri, (((1,), (0,)), ((), ()))) * 256.0
           + lax.dot_general(lo, tri, (((1,), (0,)), ((), ()))))

    pos = jnp.sum(onehot * (cum - 1.0 + off), axis=-1, keepdims=True)
    pos_ref[0] = pos.astype(jnp.int32).reshape(SEQ // 128, 128)
    off_ref[0] = off.astype(jnp.int32)
    cnt_ref[0] = ci


def _hash_sort(qk, rot_t):
    return pl.pallas_call(
        _hash_sort_body,
        grid=(GB, N_HASHES),
        in_specs=[
            pl.BlockSpec((1, SEQ, DIM), lambda b, h: (b, 0, 0)),
            pl.BlockSpec((1, DIM, N_BUCKETS // 2), lambda b, h: (h, 0, 0)),
        ],
        out_specs=[
            pl.BlockSpec((1, SEQ // 128, 128),
                         lambda b, h: (b * N_HASHES + h, 0, 0)),
            pl.BlockSpec((1, 1, N_BUCKETS),
                         lambda b, h: (b * N_HASHES + h, 0, 0)),
            pl.BlockSpec((1, 1, N_BUCKETS),
                         lambda b, h: (b * N_HASHES + h, 0, 0)),
        ],
        out_shape=[
            jax.ShapeDtypeStruct((GT, SEQ // 128, 128), jnp.int32),
            jax.ShapeDtypeStruct((GT, 1, N_BUCKETS), jnp.int32),
            jax.ShapeDtypeStruct((GT, 1, N_BUCKETS), jnp.int32),
        ],
    )(qk, rot_t)


def _permute_body(g, qk_hbm, v_hbm, pos_hbm, sqk_hbm, sv_hbm,
                  pos_v, idxg_v, bufq, bufv, sem):
    nc = 2
    wid = lax.axis_index("s") * nc + lax.axis_index("c")
    tl = wid // 2
    half = wid % 2
    b = GB * g + tl // N_HASHES
    toff = pl.multiple_of(tl * SEQ, SEQ)
    pltpu.sync_copy(pos_hbm.at[pl.ds(toff, SEQ)], pos_v)

    boff = b * SEQ

    def scat(j, _):
        base = pl.multiple_of(j * 16, 16)
        idx = pos_v[pl.ds(base, 16)]
        tok = lax.iota(jnp.int32, 16) + base
        plsc.store_scatter(idxg_v, [idx], tok + boff)
        return 0

    lax.fori_loop(0, SEQ // 16, scat, 0)

    def gat(gg, _):
        goff = pl.multiple_of((half * 4 + gg) * (4 * GCHUNK), 4 * GCHUNK)
        waits = []
        for j in range(4):
            idxs = idxg_v.at[pl.ds(goff + j * GCHUNK, GCHUNK)]
            dq = bufq.at[pl.ds(j * GCHUNK, GCHUNK)]
            dv = bufv.at[pl.ds(j * GCHUNK, GCHUNK)]
            waits.append(pltpu.async_copy(qk_hbm.at[idxs], dq, sem))
            waits.append(pltpu.async_copy(v_hbm.at[idxs], dv, sem))
        for w in waits:
            w.wait()
        pltpu.sync_copy(bufq, sqk_hbm.at[pl.ds(toff + goff, 4 * GCHUNK)])
        pltpu.sync_copy(bufv, sv_hbm.at[pl.ds(toff + goff, 4 * GCHUNK)])
        return 0

    lax.fori_loop(0, SEQ // (8 * GCHUNK), gat, 0)


def _permute(qk_flat, v_flat, pos_flat, g):
    tot = GT * SEQ
    mesh = plsc.VectorSubcoreMesh(core_axis_name="c", subcore_axis_name="s")
    fn = functools.partial(
        pl.kernel,
        out_type=[
            jax.ShapeDtypeStruct((tot, DIM), jnp.float32),
            jax.ShapeDtypeStruct((tot, DIM), jnp.float32),
        ],
        mesh=mesh,
        compiler_params=pltpu.CompilerParams(
            needs_layout_passes=False, use_tc_tiling_on_sc=False),
        scratch_types=[
            pltpu.VMEM((SEQ,), jnp.int32),
            pltpu.VMEM((SEQ,), jnp.int32),
            pltpu.VMEM((4 * GCHUNK, DIM), jnp.float32),
            pltpu.VMEM((4 * GCHUNK, DIM), jnp.float32),
            pltpu.SemaphoreType.DMA,
        ],
    )(functools.partial(_permute_body, g))
    return fn(qk_flat, v_flat, pos_flat)


def _attend_body(sqk_ref, sv_ref, off_ref, cnt_ref, so_ref, slse_ref):
    sqk = sqk_ref[0]
    sv = sv_ref[0]
    off = off_ref[0]
    cnt = cnt_ref[0]

    norms = jnp.sqrt(jnp.sum(sqk * sqk, axis=-1, keepdims=True))
    kn = sqk / jnp.maximum(norms, 1e-12)

    q = sqk.reshape(N_CHUNKS, BUCKET_SIZE, DIM)
    kc = kn.reshape(N_CHUNKS, BUCKET_SIZE, DIM)
    kwin = jnp.concatenate(
        [kc, jnp.concatenate([kc[-1:], kc[:-1]], axis=0)], axis=1)
    vc = sv.reshape(N_CHUNKS, BUCKET_SIZE, DIM)
    vwin = jnp.concatenate(
        [vc, jnp.concatenate([vc[-1:], vc[:-1]], axis=0)], axis=1)

    ji = lax.broadcasted_iota(jnp.int32, (SEQ, N_BUCKETS), 0)
    ohs = ((ji >= off) & (ji < off + cnt)).astype(jnp.float32)
    oq = ohs.reshape(N_CHUNKS, BUCKET_SIZE, N_BUCKETS)
    okw = jnp.concatenate(
        [oq, jnp.concatenate(
            [jnp.zeros((1, BUCKET_SIZE, N_BUCKETS), jnp.float32), oq[:-1]],
            axis=0)], axis=1)
    same = lax.dot_general(oq, okw, (((2,), (2,)), ((0,), (0,))))

    dots = lax.dot_general(q, kwin, (((2,), (2,)), ((0,), (0,))))
    dots = dots * (DIM ** -0.5)
    qi = lax.broadcasted_iota(jnp.int32, dots.shape, 1)
    zi = lax.broadcasted_iota(jnp.int32, dots.shape, 2)
    dots = jnp.where(qi == zi, NEG_SELF, dots)
    dots = jnp.where(same < 0.5, -jnp.finfo(jnp.float32).max, dots)

    m = jnp.max(dots, axis=-1, keepdims=True)
    e = jnp.exp(dots - m)
    s = jnp.sum(e, axis=-1, keepdims=True)
    lse = m + jnp.log(s)
    p = e * (1.0 / s)
    bo = lax.dot_general(p, vwin, (((2,), (1,)), ((0,), (0,))))
    so_ref[0] = bo.reshape(SEQ, DIM)
    slse_ref[0] = lse.reshape(SEQ, 1).reshape(SEQ // 128, 128)


def _attend(sqk, sv, offs, cnts):
    return pl.pallas_call(
        _attend_body,
        grid=(GT,),
        in_specs=[
            pl.BlockSpec((1, SEQ, DIM), lambda t: (t, 0, 0)),
            pl.BlockSpec((1, SEQ, DIM), lambda t: (t, 0, 0)),
            pl.BlockSpec((1, 1, N_BUCKETS), lambda t: (t, 0, 0)),
            pl.BlockSpec((1, 1, N_BUCKETS), lambda t: (t, 0, 0)),
        ],
        out_specs=[
            pl.BlockSpec((1, SEQ, DIM), lambda t: (t, 0, 0)),
            pl.BlockSpec((1, SEQ // 128, 128), lambda t: (t, 0, 0)),
        ],
        out_shape=[
            jax.ShapeDtypeStruct((GT, SEQ, DIM), jnp.float32),
            jax.ShapeDtypeStruct((GT, SEQ // 128, 128), jnp.float32),
        ],
    )(sqk, sv, offs, cnts)


def _unpermute_body(so_hbm, slse_hbm, pos_hbm, o_hbm, lg_hbm,
                    pos_v, gidx_v, lse_v, lgo_v, bufo, sem):
    nc = 2
    wid = lax.axis_index("s") * nc + lax.axis_index("c")
    tl = wid // 2
    half = wid % 2
    toff = pl.multiple_of(tl * SEQ, SEQ)
    pltpu.sync_copy(pos_hbm.at[pl.ds(toff, SEQ)], pos_v)
    pltpu.sync_copy(slse_hbm.at[pl.ds(toff, SEQ)], lse_v)

    hoff = half * (SEQ // 2)

    def addoff(j, _):
        base = pl.multiple_of(hoff + j * 16, 16)
        idx = pos_v[pl.ds(base, 16)]
        gidx_v[pl.ds(base, 16)] = idx + toff
        lgo_v[pl.ds(base, 16)] = plsc.load_gather(lse_v, [idx])
        return 0

    lax.fori_loop(0, SEQ // 32, addoff, 0)
    pltpu.sync_copy(lgo_v.at[pl.ds(hoff, SEQ // 2)],
                    lg_hbm.at[pl.ds(toff + hoff, SEQ // 2)])

    def gat(gg, _):
        goff = pl.multiple_of((half * 4 + gg) * (4 * GCHUNK), 4 * GCHUNK)
        waits = []
        for j in range(4):
            idxs = gidx_v.at[pl.ds(goff + j * GCHUNK, GCHUNK)]
            do = bufo.at[pl.ds(j * GCHUNK, GCHUNK)]
            waits.append(pltpu.async_copy(so_hbm.at[idxs], do, sem))
        for w in waits:
            w.wait()
        pltpu.sync_copy(bufo, o_hbm.at[pl.ds(toff + goff, 4 * GCHUNK)])
        return 0

    lax.fori_loop(0, SEQ // (8 * GCHUNK), gat, 0)


def _unpermute(so_flat, slse_flat, pos_flat):
    tot = GT * SEQ
    mesh = plsc.VectorSubcoreMesh(core_axis_name="c", subcore_axis_name="s")
    fn = functools.partial(
        pl.kernel,
        out_type=[
            jax.ShapeDtypeStruct((tot, DIM), jnp.float32),
            jax.ShapeDtypeStruct((tot,), jnp.float32),
        ],
        mesh=mesh,
        compiler_params=pltpu.CompilerParams(
            needs_layout_passes=False, use_tc_tiling_on_sc=False),
        scratch_types=[
            pltpu.VMEM((SEQ,), jnp.int32),
            pltpu.VMEM((SEQ,), jnp.int32),
            pltpu.VMEM((SEQ,), jnp.float32),
            pltpu.VMEM((SEQ,), jnp.float32),
            pltpu.VMEM((4 * GCHUNK, DIM), jnp.float32),
            pltpu.SemaphoreType.DMA,
        ],
    )(_unpermute_body)
    return fn(so_flat, slse_flat, pos_flat)


_CSEQ = 512


def _combine_body(o_ref, lg_ref, out_ref):
    o = o_ref[0]
    lg = lg_ref[0]
    m = jnp.max(lg, axis=-1, keepdims=True)
    e = jnp.exp(lg - m)
    s = jnp.sum(e, axis=-1, keepdims=True)
    p = e / s
    acc = o[0] * p[:, 0:1]
    for h in range(1, N_HASHES):
        acc = acc + o[h] * p[:, h:h + 1]
    out_ref[0] = acc


def _combine(o4, lg3t):
    return pl.pallas_call(
        _combine_body,
        grid=(GB, SEQ // _CSEQ),
        in_specs=[
            pl.BlockSpec((1, N_HASHES, _CSEQ, DIM), lambda b, s: (b, 0, s, 0)),
            pl.BlockSpec((1, _CSEQ, N_HASHES), lambda b, s: (b, s, 0)),
        ],
        out_specs=pl.BlockSpec((1, _CSEQ, DIM), lambda b, s: (b, s, 0)),
        out_shape=jax.ShapeDtypeStruct((GB, SEQ, DIM), jnp.float32),
    )(o4, lg3t)


def kernel(qk, v):
    rot = jax.random.normal(jax.random.key(42),
                            (DIM, N_HASHES, N_BUCKETS // 2), dtype=qk.dtype)
    rot_t = jnp.transpose(rot, (1, 0, 2))

    qk_flat = qk.reshape(BATCH * SEQ, DIM)
    v_flat = v.reshape(BATCH * SEQ, DIM)

    outs = []
    for g in range(NG):
        pos, offs, cnts = _hash_sort(qk[g * GB:(g + 1) * GB], rot_t)
        pos_flat = pos.reshape(GT * SEQ)
        sqk_flat, sv_flat = _permute(qk_flat, v_flat, pos_flat, g)
        so, slse = _attend(sqk_flat.reshape(GT, SEQ, DIM),
                           sv_flat.reshape(GT, SEQ, DIM),
                           offs, cnts)
        o_flat, lg_flat = _unpermute(so.reshape(GT * SEQ, DIM),
                                     slse.reshape(GT * SEQ),
                                     pos_flat)
        lg3t = jnp.transpose(lg_flat.reshape(GB, N_HASHES, SEQ), (0, 2, 1))
        outs.append(_combine(o_flat.reshape(GB, N_HASHES, SEQ, DIM), lg3t))
    return jnp.concatenate(outs, axis=0)


# speedup vs baseline: 9.7928x; 1.0398x over previous
import functools

import jax
import jax.numpy as jnp
from jax import lax
from jax.experimental import pallas as pl
from jax.experimental.pallas import tpu as pltpu
from jax.experimental.pallas import tpu_sc as plsc

BATCH = 8
SEQ = 4096
DIM = 64
N_HASHES = 8
BUCKET_SIZE = 64
N_BUCKETS = SEQ // BUCKET_SIZE
N_CHUNKS = SEQ // BUCKET_SIZE
TASKS = BATCH * N_HASHES
NEG_SELF = -10000.0
GCHUNK = 128
NG = 2
GB = BATCH // NG
GT = TASKS // NG


def _hash_sort_body(qk_ref, rot_ref, pos_ref, off_ref, cnt_ref):
    qk = qk_ref[0]
    rot = rot_ref[0]
    r = lax.dot_general(qk, rot, (((1,), (0,)), ((), ())),
                        preferred_element_type=jnp.float32)
    r2 = jnp.concatenate([r, -r], axis=-1)
    m = jnp.max(r2, axis=-1, keepdims=True)
    col = lax.broadcasted_iota(jnp.int32, r2.shape, 1)
    bucket = jnp.min(jnp.where(r2 == m, col, N_BUCKETS), axis=-1,
                     keepdims=True)

    onehot = (bucket == lax.broadcasted_iota(
        jnp.int32, (SEQ, N_BUCKETS), 1)).astype(jnp.float32)

    cum = onehot
    k = 1
    while k < SEQ:
        cum = cum + jnp.concatenate(
            [jnp.zeros((k, N_BUCKETS), jnp.float32), cum[:-k]], axis=0)
        k *= 2

    counts = jnp.sum(onehot, axis=0, keepdims=True)
    ci = counts.astype(jnp.int32)
    hi = (ci >> 8).astype(jnp.float32)
    lo = (ci & 255).astype(jnp.float32)
    tri = (lax.broadcasted_iota(jnp.int32, (N_BUCKETS, N_BUCKETS), 0)
           < lax.broadcasted_iota(
               jnp.int32, (N_BUCKETS, N_BUCKETS), 1)).astype(jnp.float32)
    off = (lax.dot_general(hi, tri, (((1,), (0,)), ((), ()))) * 256.0
           + lax.dot_general(lo, tri, (((1,), (0,)), ((), ()))))

    pos = jnp.sum(onehot * (cum - 1.0 + off), axis=-1, keepdims=True)
    pos_ref[0] = pos.astype(jnp.int32).reshape(SEQ // 128, 128)
    off_ref[0] = off.astype(jnp.int32)
    cnt_ref[0] = ci


def _hash_sort(qk, rot_t):
    return pl.pallas_call(
        _hash_sort_body,
        grid=(GB, N_HASHES),
        in_specs=[
            pl.BlockSpec((1, SEQ, DIM), lambda b, h: (b, 0, 0)),
            pl.BlockSpec((1, DIM, N_BUCKETS // 2), lambda b, h: (h, 0, 0)),
        ],
        out_specs=[
            pl.BlockSpec((1, SEQ // 128, 128),
                         lambda b, h: (b * N_HASHES + h, 0, 0)),
            pl.BlockSpec((1, 1, N_BUCKETS),
                         lambda b, h: (b * N_HASHES + h, 0, 0)),
            pl.BlockSpec((1, 1, N_BUCKETS),
                         lambda b, h: (b * N_HASHES + h, 0, 0)),
        ],
        out_shape=[
            jax.ShapeDtypeStruct((GT, SEQ // 128, 128), jnp.int32),
            jax.ShapeDtypeStruct((GT, 1, N_BUCKETS), jnp.int32),
            jax.ShapeDtypeStruct((GT, 1, N_BUCKETS), jnp.int32),
        ],
    )(qk, rot_t)


def _permute_body(g, qk_hbm, v_hbm, pos_hbm, sqk_hbm, sv_hbm,
                  pos_v, idxg_v, bufq, bufv, sem):
    nc = 2
    wid = lax.axis_index("s") * nc + lax.axis_index("c")
    tl = wid // 2
    half = wid % 2
    b = GB * g + tl // N_HASHES
    toff = pl.multiple_of(tl * SEQ, SEQ)
    pltpu.sync_copy(pos_hbm.at[pl.ds(toff, SEQ)], pos_v)

    boff = b * SEQ

    def scat(j, _):
        base = pl.multiple_of(j * 16, 16)
        idx = pos_v[pl.ds(base, 16)]
        tok = lax.iota(jnp.int32, 16) + base
        plsc.store_scatter(idxg_v, [idx], tok + boff)
        return 0

    lax.fori_loop(0, SEQ // 16, scat, 0)

    def gat(gg, _):
        goff = pl.multiple_of((half * 4 + gg) * (4 * GCHUNK), 4 * GCHUNK)
        waits = []
        for j in range(4):
            idxs = idxg_v.at[pl.ds(goff + j * GCHUNK, GCHUNK)]
            dq = bufq.at[pl.ds(j * GCHUNK, GCHUNK)]
            dv = bufv.at[pl.ds(j * GCHUNK, GCHUNK)]
            waits.append(pltpu.async_copy(qk_hbm.at[idxs], dq, sem))
            waits.append(pltpu.async_copy(v_hbm.at[idxs], dv, sem))
        for w in waits:
            w.wait()
        pltpu.sync_copy(bufq, sqk_hbm.at[pl.ds(toff + goff, 4 * GCHUNK)])
        pltpu.sync_copy(bufv, sv_hbm.at[pl.ds(toff + goff, 4 * GCHUNK)])
        return 0

    lax.fori_loop(0, SEQ // (8 * GCHUNK), gat, 0)


def _permute(qk_flat, v_flat, pos_flat, g):
    tot = GT * SEQ
    mesh = plsc.VectorSubcoreMesh(core_axis_name="c", subcore_axis_name="s")
    fn = functools.partial(
        pl.kernel,
        out_type=[
            jax.ShapeDtypeStruct((tot, DIM), jnp.float32),
            jax.ShapeDtypeStruct((tot, DIM), jnp.float32),
        ],
        mesh=mesh,
        compiler_params=pltpu.CompilerParams(
            needs_layout_passes=False, use_tc_tiling_on_sc=False),
        scratch_types=[
            pltpu.VMEM((SEQ,), jnp.int32),
            pltpu.VMEM((SEQ,), jnp.int32),
            pltpu.VMEM((4 * GCHUNK, DIM), jnp.float32),
            pltpu.VMEM((4 * GCHUNK, DIM), jnp.float32),
            pltpu.SemaphoreType.DMA,
        ],
    )(functools.partial(_permute_body, g))
    return fn(qk_flat, v_flat, pos_flat)


def _attend_body(sqk_ref, sv_ref, off_ref, cnt_ref, so_ref, slse_ref):
    sqk = sqk_ref[0]
    sv = sv_ref[0]
    off = off_ref[0]
    cnt = cnt_ref[0]

    norms = jnp.sqrt(jnp.sum(sqk * sqk, axis=-1, keepdims=True))
    kn = sqk / jnp.maximum(norms, 1e-12)

    q = sqk.reshape(N_CHUNKS, BUCKET_SIZE, DIM)
    kc = kn.reshape(N_CHUNKS, BUCKET_SIZE, DIM)
    kwin = jnp.concatenate(
        [kc, jnp.concatenate([kc[-1:], kc[:-1]], axis=0)], axis=1)
    vc = sv.reshape(N_CHUNKS, BUCKET_SIZE, DIM)
    vwin = jnp.concatenate(
        [vc, jnp.concatenate([vc[-1:], vc[:-1]], axis=0)], axis=1)

    ji = lax.broadcasted_iota(jnp.int32, (SEQ, N_BUCKETS), 0)
    ohs = ((ji >= off) & (ji < off + cnt)).astype(jnp.float32)
    oq = ohs.reshape(N_CHUNKS, BUCKET_SIZE, N_BUCKETS)
    okw = jnp.concatenate(
        [oq, jnp.concatenate(
            [jnp.zeros((1, BUCKET_SIZE, N_BUCKETS), jnp.float32), oq[:-1]],
            axis=0)], axis=1)
    same = lax.dot_general(oq, okw, (((2,), (2,)), ((0,), (0,))))

    dots = lax.dot_general(q, kwin, (((2,), (2,)), ((0,), (0,))))
    dots = dots * (DIM ** -0.5)
    qi = lax.broadcasted_iota(jnp.int32, dots.shape, 1)
    zi = lax.broadcasted_iota(jnp.int32, dots.shape, 2)
    dots = jnp.where(qi == zi, NEG_SELF, dots)
    dots = jnp.where(same < 0.5, -jnp.finfo(jnp.float32).max, dots)

    m = jnp.max(dots, axis=-1, keepdims=True)
    e = jnp.exp(dots - m)
    s = jnp.sum(e, axis=-1, keepdims=True)
    lse = m + jnp.log(s)
    p = e * (1.0 / s)
    bo = lax.dot_general(p, vwin, (((2,), (1,)), ((0,), (0,))))
    so_ref[0] = bo.reshape(SEQ, DIM)
    slse_ref[0] = lse.reshape(SEQ, 1).reshape(SEQ // 128, 128)


def _attend(sqk, sv, offs, cnts):
    return pl.pallas_call(
        _attend_body,
        grid=(GT,),
        in_specs=[
            pl.BlockSpec((1, SEQ, DIM), lambda t: (t, 0, 0)),
            pl.BlockSpec((1, SEQ, DIM), lambda t: (t, 0, 0)),
            pl.BlockSpec((1, 1, N_BUCKETS), lambda t: (t, 0, 0)),
            pl.BlockSpec((1, 1, N_BUCKETS), lambda t: (t, 0, 0)),
        ],
        out_specs=[
            pl.BlockSpec((1, SEQ, DIM), lambda t: (t, 0, 0)),
            pl.BlockSpec((1, SEQ // 128, 128), lambda t: (t, 0, 0)),
        ],
        out_shape=[
            jax.ShapeDtypeStruct((GT, SEQ, DIM), jnp.float32),
            jax.ShapeDtypeStruct((GT, SEQ // 128, 128), jnp.float32),
        ],
    )(sqk, sv, offs, cnts)


def _unpermute_body(so_hbm, slse_hbm, pos_hbm, o_hbm, lg_hbm,
                    pos_v, gidx_v, lse_v, lgo_v, bufo, sem):
    nc = 2
    wid = lax.axis_index("s") * nc + lax.axis_index("c")
    tl = wid // 2
    half = wid % 2
    toff = pl.multiple_of(tl * SEQ, SEQ)
    pltpu.sync_copy(pos_hbm.at[pl.ds(toff, SEQ)], pos_v)
    pltpu.sync_copy(slse_hbm.at[pl.ds(toff, SEQ)], lse_v)

    hoff = half * (SEQ // 2)

    def addoff(j, _):
        base = pl.multiple_of(hoff + j * 16, 16)
        idx = pos_v[pl.ds(base, 16)]
        gidx_v[pl.ds(base, 16)] = idx + toff
        lgo_v[pl.ds(base, 16)] = plsc.load_gather(lse_v, [idx])
        return 0

    lax.fori_loop(0, SEQ // 32, addoff, 0)
    pltpu.sync_copy(lgo_v.at[pl.ds(hoff, SEQ // 2)],
                    lg_hbm.at[pl.ds(toff + hoff, SEQ // 2)])

    def gat(gg, _):
        goff = pl.multiple_of((half * 4 + gg) * (4 * GCHUNK), 4 * GCHUNK)
        waits = []
        for j in range(4):
            idxs = gidx_v.at[pl.ds(goff + j * GCHUNK, GCHUNK)]
            do = bufo.at[pl.ds(j * GCHUNK, GCHUNK)]
            waits.append(pltpu.async_copy(so_hbm.at[idxs], do, sem))
        for w in waits:
            w.wait()
        pltpu.sync_copy(bufo, o_hbm.at[pl.ds(toff + goff, 4 * GCHUNK)])
        return 0

    lax.fori_loop(0, SEQ // (8 * GCHUNK), gat, 0)


def _unpermute(so_flat, slse_flat, pos_flat):
    tot = GT * SEQ
    mesh = plsc.VectorSubcoreMesh(core_axis_name="c", subcore_axis_name="s")
    fn = functools.partial(
        pl.kernel,
        out_type=[
            jax.ShapeDtypeStruct((tot, DIM), jnp.float32),
            jax.ShapeDtypeStruct((tot,), jnp.float32),
        ],
        mesh=mesh,
        compiler_params=pltpu.CompilerParams(
            needs_layout_passes=False, use_tc_tiling_on_sc=False),
        scratch_types=[
            pltpu.VMEM((SEQ,), jnp.int32),
            pltpu.VMEM((SEQ,), jnp.int32),
            pltpu.VMEM((SEQ,), jnp.float32),
            pltpu.VMEM((SEQ,), jnp.float32),
            pltpu.VMEM((4 * GCHUNK, DIM), jnp.float32),
            pltpu.SemaphoreType.DMA,
        ],
    )(_unpermute_body)
    return fn(so_flat, slse_flat, pos_flat)


_CSEQ = 512


def _combine_body(o_ref, lg_ref, out_ref):
    o = o_ref[0]
    lg = lg_ref[0]
    m = jnp.max(lg, axis=-1, keepdims=True)
    e = jnp.exp(lg - m)
    s = jnp.sum(e, axis=-1, keepdims=True)
    p = e / s
    acc = o[0] * p[:, 0:1]
    for h in range(1, N_HASHES):
        acc = acc + o[h] * p[:, h:h + 1]
    out_ref[0] = acc


def _combine(o4, lg3t):
    return pl.pallas_call(
        _combine_body,
        grid=(GB, SEQ // _CSEQ),
        in_specs=[
            pl.BlockSpec((1, N_HASHES, _CSEQ, DIM), lambda b, s: (b, 0, s, 0)),
            pl.BlockSpec((1, _CSEQ, N_HASHES), lambda b, s: (b, s, 0)),
        ],
        out_specs=pl.BlockSpec((1, _CSEQ, DIM), lambda b, s: (b, s, 0)),
        out_shape=jax.ShapeDtypeStruct((GB, SEQ, DIM), jnp.float32),
    )(o4, lg3t)


def kernel(qk, v):
    rot = jax.random.normal(jax.random.key(42),
                            (DIM, N_HASHES, N_BUCKETS // 2), dtype=qk.dtype)
    rot_t = jnp.transpose(rot, (1, 0, 2))

    qk_flat = qk.reshape(BATCH * SEQ, DIM)
    v_flat = v.reshape(BATCH * SEQ, DIM)

    outs = []
    for g in range(NG):
        pos, offs, cnts = _hash_sort(qk[g * GB:(g + 1) * GB], rot_t)
        pos_flat = pos.reshape(GT * SEQ)
        sqk_flat, sv_flat = _permute(qk_flat, v_flat, pos_flat, g)
        so, slse = _attend(sqk_flat.reshape(GT, SEQ, DIM),
                           sv_flat.reshape(GT, SEQ, DIM),
                           offs, cnts)
        o_flat, lg_flat = _unpermute(so.reshape(GT * SEQ, DIM),
                                     slse.reshape(GT * SEQ),
                                     pos_flat)
        lg3t = jnp.transpose(lg_flat.reshape(GB, N_HASHES, SEQ), (0, 2, 1))
        outs.append(_combine(o_flat.reshape(GB, N_HASHES, SEQ, DIM), lg3t))
    return jnp.concatenate(outs, axis=0)
